# baseline (speedup 1.0000x reference)
import jax
import jax.numpy as jnp
from jax.experimental import pallas as pl
from jax.experimental.pallas import tpu as pltpu

_NS = 128


def _diag_body(x_ref, o_ref):
    s = jnp.sum(x_ref[0], axis=1, keepdims=True)
    o_ref[0] = jnp.broadcast_to(s[:, :, None], o_ref.shape[1:])


def kernel(x):
    B, L, F = x.shape
    xt = x.reshape(B, F, L)
    return pl.pallas_call(
        _diag_body,
        grid=(B,),
        in_specs=[pl.BlockSpec((1, F, L), lambda b: (b, 0, 0))],
        out_specs=pl.BlockSpec((1, F, _NS, _NS), lambda b: (b, 0, 0, 0)),
        out_shape=jax.ShapeDtypeStruct((B, F, _NS, _NS), jnp.float32),
        compiler_params=pltpu.CompilerParams(
            dimension_semantics=("parallel",),
        ),
    )(xt)


# speedup vs baseline: 2.3702x; 2.3702x over previous
import jax
import jax.numpy as jnp
from jax.experimental import pallas as pl
from jax.experimental.pallas import tpu as pltpu

_NS = 128


def _mt_body(x_ref, o_ref):
    data = x_ref[0]
    F, L = data.shape
    mn = jnp.min(data, axis=1, keepdims=True)
    mx = jnp.max(data, axis=1, keepdims=True)
    denom = jnp.maximum(mx - mn, 1e-8)
    xn = (data - mn) / denom
    scaled = jnp.clip(xn * (_NS - 1), 0.0, _NS - 1)
    idx = jnp.floor(scaled).astype(jnp.int32)
    idx_n = jnp.concatenate([idx[:, 1:], idx[:, :1]], axis=1)
    lane = jax.lax.broadcasted_iota(jnp.int32, (1, L), 1)
    idx_a = jnp.where(lane < (L - 1), idx, -1).astype(jnp.int8)
    idx_n = idx_n.astype(jnp.int8)
    states = jax.lax.broadcasted_iota(jnp.int8, (_NS, L), 0)
    one = jnp.float8_e4m3fn(1.0)
    zero = jnp.float8_e4m3fn(0.0)
    for f in range(F):
        a = jnp.where(idx_a[f : f + 1, :] == states, one, zero)
        b = jnp.where(idx_n[f : f + 1, :] == states, one, zero)
        c = jax.lax.dot_general(
            a,
            b,
            (((1,), (1,)), ((), ())),
            preferred_element_type=jnp.float32,
        )
        rs = jnp.maximum(jnp.sum(c, axis=1, keepdims=True), 1e-8)
        o_ref[0, f] = c / rs


def kernel(x):
    B, L, F = x.shape
    _GB = 4
    xt = jnp.transpose(x, (0, 2, 1)).reshape(B // _GB, _GB * F, L)
    out = pl.pallas_call(
        _mt_body,
        grid=(B // _GB,),
        in_specs=[pl.BlockSpec((1, _GB * F, L), lambda b: (b, 0, 0))],
        out_specs=pl.BlockSpec((1, _GB * F, _NS, _NS), lambda b: (b, 0, 0, 0)),
        out_shape=jax.ShapeDtypeStruct((B // _GB, _GB * F, _NS, _NS), jnp.float32),
        compiler_params=pltpu.CompilerParams(
            dimension_semantics=("parallel",),
        ),
    )(xt)
    return out.reshape(B, F, _NS, _NS)
